# baseline (speedup 1.0000x reference)
import jax
import jax.numpy as jnp
from jax.experimental import pallas as pl
from jax.experimental.pallas import tpu as pltpu

E = 8
D = 768


def _moe_body(onehot_ref, eid_ref, x_ref, W_ref, b_ref, out_ref, xb_ref):
    e = pl.program_id(0)

    @pl.when(e == 0)
    def _():
        out_ref[...] = jnp.dot(onehot_ref[...], b_ref[...],
                               preferred_element_type=jnp.float32)
        xb_ref[...] = x_ref[...].astype(jnp.bfloat16)

    mask = (eid_ref[...] == e).astype(jnp.bfloat16)
    xm = xb_ref[...] * mask
    out_ref[...] = out_ref[...] + jnp.dot(
        xm, W_ref[0].astype(jnp.bfloat16),
        preferred_element_type=jnp.float32)


def kernel(x, curr_video_id, W, b):
    B = x.shape[0]
    eid = curr_video_id.astype(jnp.int32)
    onehot = jax.nn.one_hot(eid, E, dtype=x.dtype)

    return pl.pallas_call(
        _moe_body,
        grid=(E,),
        in_specs=[
            pl.BlockSpec((B, E), lambda e: (0, 0)),
            pl.BlockSpec((B, 1), lambda e: (0, 0)),
            pl.BlockSpec((B, D), lambda e: (0, 0)),
            pl.BlockSpec((1, D, D), lambda e: (e, 0, 0)),
            pl.BlockSpec((E, D), lambda e: (0, 0)),
        ],
        out_specs=pl.BlockSpec((B, D), lambda e: (0, 0)),
        out_shape=jax.ShapeDtypeStruct((B, D), x.dtype),
        scratch_shapes=[pltpu.VMEM((B, D), jnp.bfloat16)],
    )(onehot, eid.reshape(B, 1), x, W, b)
